# baseline (speedup 1.0000x reference)
import functools
import numpy as np
import jax
from jax import lax
import jax.numpy as jnp
from jax.experimental import pallas as pl
from jax.experimental.pallas import tpu as pltpu
from jax.experimental.pallas import tpu_sc as plsc

B, N, IN_DIM, DEP = 8, 64, 128, 8
K = N * DEP
ROWS = B * N
NC, NS, L = 2, 16, 16
RPW = ROWS // (NC * NS)
NCH = N // L

_mesh = plsc.VectorSubcoreMesh(core_axis_name="c", subcore_axis_name="s")


def _sc_body(adj_hbm, w2_hbm, s_hbm, adjv, sv, wv):
    wid = lax.axis_index("s") * NC + lax.axis_index("c")
    base = wid * RPW
    pltpu.sync_copy(adj_hbm.at[pl.ds(base * K, RPW * K)], adjv)
    pltpu.sync_copy(w2_hbm, wv)
    lanes = lax.iota(jnp.int32, L)
    lanesf = lanes.astype(jnp.float32)
    w2vec = wv[...]

    def row_body(row, carry):
        roff = row * K
        cacc = [jnp.zeros((L,), jnp.float32) for _ in range(NCH)]
        vacc = [jnp.zeros((L,), jnp.float32) for _ in range(NCH)]
        radd = jnp.zeros((L,), jnp.float32)
        for d in range(DEP):
            w2d = w2vec[d]
            dtot = jnp.zeros((L,), jnp.float32)
            for ch in range(NCH):
                idx = roff + (lanes + L * ch) * DEP + d
                g = plsc.load_gather(adjv, [idx])
                nz = (g != 0.0).astype(jnp.float32)
                cacc[ch] = cacc[ch] + nz
                vacc[ch] = vacc[ch] + g * w2d
                dtot = dtot + nz
            rd = jnp.sum(dtot)
            radd = radd + jnp.where(lanesf == float(d), rd, 0.0)
        cacc[0] = cacc[0] + radd
        for ch in range(NCH):
            sv[pl.ds(row * N + ch * L, L)] = cacc[ch] * jnp.exp(vacc[ch])
        return carry

    lax.fori_loop(0, RPW, row_body, 0)
    pltpu.sync_copy(sv, s_hbm.at[pl.ds(base * N, RPW * N)])


_sc_weights = functools.partial(
    pl.kernel,
    out_type=jax.ShapeDtypeStruct((ROWS * N,), jnp.float32),
    mesh=_mesh,
    compiler_params=pltpu.CompilerParams(needs_layout_passes=False),
    scratch_types=[
        pltpu.VMEM((RPW * K,), jnp.float32),
        pltpu.VMEM((RPW * N,), jnp.float32),
        pltpu.VMEM((L,), jnp.float32),
    ],
)(_sc_body)

_CONTRACT_LAST = (((1,), (1,)), ((), ()))


def _tc_body(f_ref, aspect_ref, s_ref, w_ref, out_ref):
    f = f_ref[0]
    s = s_ref[0]
    w1row = w_ref[:, :IN_DIM]

    hi = jax.lax.Precision.HIGHEST
    urow = lax.dot_general(w1row, f, _CONTRACT_LAST, precision=hi)
    st = s * jnp.exp(urow)
    num = jnp.dot(st, f, precision=hi)
    den = jnp.sum(st, axis=1, keepdims=True)
    agg = num / den

    any_valid = jnp.any(s > 0.0, axis=1, keepdims=True)
    upd = (aspect_ref[0] != 0) & any_valid
    out_ref[0] = jnp.where(upd, agg, f)


def kernel(features, aspect_onehot, adj_matrix, W):
    adj_flat = adj_matrix.reshape(ROWS * K)
    w2p = jnp.pad(W[0, IN_DIM:IN_DIM + DEP], (0, L - DEP))
    s = _sc_weights(adj_flat, w2p).reshape(B, N, N)
    aspect3 = aspect_onehot.reshape(B, N, 1).astype(jnp.int32)
    return pl.pallas_call(
        _tc_body,
        grid=(B,),
        in_specs=[
            pl.BlockSpec((1, N, IN_DIM), lambda b: (b, 0, 0)),
            pl.BlockSpec((1, N, 1), lambda b: (b, 0, 0)),
            pl.BlockSpec((1, N, N), lambda b: (b, 0, 0)),
            pl.BlockSpec((1, IN_DIM + DEP + IN_DIM), lambda b: (0, 0)),
        ],
        out_specs=pl.BlockSpec((1, N, IN_DIM), lambda b: (b, 0, 0)),
        out_shape=jax.ShapeDtypeStruct((B, N, IN_DIM), jnp.float32),
    )(features, aspect3, s, W)


# speedup vs baseline: 3.8563x; 3.8563x over previous
import numpy as np
import jax
import jax.numpy as jnp
from jax.experimental import pallas as pl

B, N, IN_DIM, DEP = 8, 64, 128, 8
K = N * DEP

_kk = np.arange(K)
_MULT_SEL = ((_kk // DEP)[:, None] == np.arange(N)[None, :]).astype(np.float32) + (
    (_kk % DEP)[:, None] == np.arange(N)[None, :]
).astype(np.float32)
_DEP_SEL = ((_kk // DEP)[:, None] == np.arange(N)[None, :]).astype(np.float32)
_DEP_MOD = ((_kk % DEP)[:, None] == np.arange(DEP)[None, :]).astype(np.float32)

_CONTRACT_LAST = (((1,), (1,)), ((), ()))


def _body(f_ref, aspect_ref, adj_ref, w_ref, msel_ref, dsel_ref, dmod_ref, out_ref):
    f = f_ref[0]
    adj2 = adj_ref[0]
    w1row = w_ref[:, :IN_DIM]
    w2row = w_ref[:, IN_DIM:IN_DIM + DEP]

    hi = jax.lax.Precision.HIGHEST
    nz = (adj2 != 0.0).astype(jnp.float32)
    mult = jnp.dot(nz, msel_ref[...], precision=hi)

    w2col = jax.lax.dot_general(dmod_ref[...], w2row, _CONTRACT_LAST,
                                precision=hi)
    vsel = dsel_ref[...] * w2col
    v = jnp.dot(adj2, vsel, precision=hi)

    urow = jax.lax.dot_general(w1row, f, _CONTRACT_LAST, precision=hi)

    e = v + urow
    validf = mult > 0.0
    emax = jnp.max(jnp.where(validf, e, -1e30), axis=1, keepdims=True)
    w = jnp.where(validf, mult * jnp.exp(e - emax), 0.0)
    z = jnp.sum(w, axis=1, keepdims=True)
    agg = jnp.dot(w / z, f, precision=hi)

    any_valid = jnp.any(validf, axis=1, keepdims=True)
    upd = (aspect_ref[0] != 0) & any_valid
    out_ref[0] = jnp.where(upd, agg, f)


def kernel(features, aspect_onehot, adj_matrix, W):
    adj2 = adj_matrix.reshape(B, N, K)
    aspect3 = aspect_onehot.reshape(B, N, 1).astype(jnp.int32)
    return pl.pallas_call(
        _body,
        grid=(B,),
        in_specs=[
            pl.BlockSpec((1, N, IN_DIM), lambda b: (b, 0, 0)),
            pl.BlockSpec((1, N, 1), lambda b: (b, 0, 0)),
            pl.BlockSpec((1, N, K), lambda b: (b, 0, 0)),
            pl.BlockSpec((1, IN_DIM + DEP + IN_DIM), lambda b: (0, 0)),
            pl.BlockSpec((K, N), lambda b: (0, 0)),
            pl.BlockSpec((K, N), lambda b: (0, 0)),
            pl.BlockSpec((K, DEP), lambda b: (0, 0)),
        ],
        out_specs=pl.BlockSpec((1, N, IN_DIM), lambda b: (b, 0, 0)),
        out_shape=jax.ShapeDtypeStruct((B, N, IN_DIM), jnp.float32),
    )(features, aspect3, adj2, W,
      jnp.asarray(_MULT_SEL), jnp.asarray(_DEP_SEL), jnp.asarray(_DEP_MOD))
